# Initial kernel scaffold; baseline (speedup 1.0000x reference)
import jax
import jax.numpy as jnp
from jax.experimental import pallas as pl


def kernel(output, labels):
    raise NotImplementedError("write your pallas kernel here")


# speedup vs baseline: 2.9492x; 2.9492x over previous
import jax
import jax.numpy as jnp
from jax.experimental import pallas as pl
from jax.experimental.pallas import tpu as pltpu

_ROWS = 1024
_COLS = 100000
_RANK = 100
_BLK = 8
_CW = 128
_NCHUNK = 782
_PADW = _NCHUNK * _CW
_NEG = float("-inf")


def _body(x_ref, lab_ref, vals_ref, pos_ref):
    lane = jax.lax.broadcasted_iota(jnp.int32, (_BLK, _CW), 1)
    lanec = jax.lax.broadcasted_iota(jnp.int32, (_BLK, _NCHUNK), 1)
    row8 = jax.lax.broadcasted_iota(jnp.int32, (_BLK, 1), 0)

    def init_c(c, cmx):
        v = x_ref[:, pl.ds(c * _CW, _CW)]
        m_c = jnp.max(v, axis=1, keepdims=True)
        return jnp.where(lanec == c, m_c, cmx)

    cmx = jax.lax.fori_loop(
        jnp.int32(0), jnp.int32(_NCHUNK), init_c,
        jnp.full((_BLK, _NCHUNK), _NEG, jnp.float32))

    def extract(i, carry):
        vals_acc, idx_acc, cmx = carry
        m = jnp.max(cmx, axis=1, keepdims=True)
        c_sel = jnp.max(jnp.where(cmx == m, lanec, -1), axis=1,
                        keepdims=True)
        nm = jnp.full((_BLK, 1), _NEG, jnp.float32)
        gv = jnp.zeros((_BLK, 1), jnp.int32)
        for r in range(_BLK):
            rmask = row8 == r
            c_r = jnp.max(jnp.where(rmask, c_sel, -1))
            m_r = jnp.max(jnp.where(rmask, m, _NEG))
            v = x_ref[pl.ds(r, 1), pl.ds(c_r * _CW, _CW)]
            l1 = lane[0:1, :]
            li = jnp.max(jnp.where(v == m_r, l1, -1))
            newv = jnp.where(l1 == li, _NEG, v)
            x_ref[pl.ds(r, 1), pl.ds(c_r * _CW, _CW)] = newv
            nm = jnp.where(rmask, jnp.max(newv), nm)
            gv = jnp.where(rmask, c_r * _CW + li, gv)
        cmx = jnp.where(lanec == c_sel, nm, cmx)
        vals_acc = jnp.where(lane == (_RANK - 1) - i, m, vals_acc)
        idx_acc = jnp.where(lane == (_RANK - 1) - i, gv, idx_acc)
        return vals_acc, idx_acc, cmx

    vals_acc, idx_acc, _ = jax.lax.fori_loop(
        jnp.int32(0), jnp.int32(_RANK), extract,
        (jnp.full((_BLK, _CW), _NEG, jnp.float32),
         jnp.full((_BLK, _CW), -1, jnp.int32),
         cmx))

    labs = lab_ref[:, :]
    lw = idx_acc == labs
    pos = jnp.max(jnp.where(lw, lane, -1), axis=1, keepdims=True)
    has = pos >= 0
    pos_ref[:, :] = jnp.where(has, pos, 0)

    lv = jnp.full((_BLK, 1), _NEG, jnp.float32)
    for r in range(_BLK):
        rmask = row8 == r
        lab_r = jnp.max(jnp.where(rmask, labs, -1))
        lc = lab_r // _CW
        lo = lab_r - lc * _CW
        v = x_ref[pl.ds(r, 1), pl.ds(lc * _CW, _CW)]
        lv_r = jnp.max(jnp.where(lane[0:1, :] == lo, v, _NEG))
        lv = jnp.where(rmask, lv_r, lv)
    vals = jnp.where(jnp.logical_and(lane == 0, jnp.logical_not(has)),
                     lv, vals_acc)
    vals_ref[:, :] = vals[:, :_RANK]


def kernel(output, labels):
    x = jnp.pad(output, ((0, 0), (0, _PADW - _COLS)),
                constant_values=-jnp.inf)
    lab32 = labels.astype(jnp.int32).reshape(_ROWS, 1)
    imap = lambda i: (i, i * 0)
    vals, pos = pl.pallas_call(
        _body,
        grid=(_ROWS // _BLK,),
        in_specs=[
            pl.BlockSpec((_BLK, _PADW), imap),
            pl.BlockSpec((_BLK, 1), imap),
        ],
        out_specs=[
            pl.BlockSpec((_BLK, _RANK), imap),
            pl.BlockSpec((_BLK, 1), imap),
        ],
        out_shape=[
            jax.ShapeDtypeStruct((_ROWS, _RANK), jnp.float32),
            jax.ShapeDtypeStruct((_ROWS, 1), jnp.int32),
        ],
    )(x, lab32)
    return vals, pos.reshape(_ROWS).astype(labels.dtype)
